# baseline (speedup 1.0000x reference)
import functools

import jax
import jax.numpy as jnp
from jax.experimental import pallas as pl
from jax.experimental.pallas import tpu as pltpu


def _ceil_to(x, m):
    return (x + m - 1) // m * m


def _mm_body(x_ref, w_ref, b_ref, o_ref):
    o_ref[...] = (
        jnp.dot(x_ref[...], w_ref[...], preferred_element_type=jnp.float32)
        + b_ref[...]
    )


@functools.partial(jax.jit, static_argnames=("bm",))
def _mm(x, w, b, bm=512):
    M, K = x.shape
    H = w.shape[1]
    Kp = _ceil_to(K, 128)
    Hp = _ceil_to(H, 128)
    Mp = _ceil_to(M, bm)
    xp = jnp.pad(x, ((0, Mp - M), (0, Kp - K)))
    wp = jnp.pad(w, ((0, Kp - K), (0, Hp - H)))
    bp = jnp.pad(b, (0, Hp - H)).reshape(1, Hp)
    out = pl.pallas_call(
        _mm_body,
        grid=(Mp // bm,),
        in_specs=[
            pl.BlockSpec((bm, Kp), lambda i: (i, 0)),
            pl.BlockSpec((Kp, Hp), lambda i: (0, 0)),
            pl.BlockSpec((1, Hp), lambda i: (0, 0)),
        ],
        out_specs=pl.BlockSpec((bm, Hp), lambda i: (i, 0)),
        out_shape=jax.ShapeDtypeStruct((Mp, Hp), jnp.float32),
    )(xp, wp, bp)
    return out[:M, :H]


def _bn(x, gamma, beta):
    m = jnp.mean(x, axis=0, keepdims=True)
    v = jnp.var(x, axis=0, keepdims=True)
    return (x - m) * jax.lax.rsqrt(v + 1e-5) * gamma + beta


def _gated_layer(h, e, src, dst, snorm_n, snorm_e, lp):
    h_in, e_in = h, e
    Ah = _mm(h, lp['A'][0], lp['A'][1])
    Bh = _mm(h, lp['B'][0], lp['B'][1])
    Ce = _mm(e, lp['C'][0], lp['C'][1], bm=1024)
    Dh = _mm(h, lp['D'][0], lp['D'][1])
    Eh = _mm(h, lp['E'][0], lp['E'][1])
    e_new = Ce + jnp.take(Dh, src, axis=0) + jnp.take(Eh, dst, axis=0)
    sigma = jax.nn.sigmoid(e_new)
    num = jax.ops.segment_sum(sigma * jnp.take(Bh, src, axis=0), dst,
                              num_segments=h.shape[0])
    den = jax.ops.segment_sum(sigma, dst, num_segments=h.shape[0])
    h_new = Ah + num / (den + 1e-6)
    h_new = h_new * snorm_n
    e_new = e_new * snorm_e
    h_new = _bn(h_new, lp['bn_h'][0], lp['bn_h'][1])
    e_new = _bn(e_new, lp['bn_e'][0], lp['bn_e'][1])
    h_new = jax.nn.relu(h_new)
    e_new = jax.nn.relu(e_new)
    return h_in + h_new, e_in + e_new


def kernel(nodes_feat, edges_feat, nodes_num_norm_sqrt, edges_num_norm_sqrt,
           edge_index, params):
    src = edge_index[0]
    dst = edge_index[1]
    h = _mm(nodes_feat, params['emb_h'][0], params['emb_h'][1])
    e = _mm(edges_feat, params['emb_e'][0], params['emb_e'][1], bm=1024)
    for lp in params['layers']:
        h, e = _gated_layer(h, e, src, dst, nodes_num_norm_sqrt,
                            edges_num_norm_sqrt, lp)
    hg = jnp.mean(h, axis=0, keepdims=True)
    y = hg
    nmlp = len(params['mlp'])
    for j, (w, b) in enumerate(params['mlp']):
        y = _mm(y, w, b, bm=8)
        if j < nmlp - 1:
            y = jax.nn.relu(y)
    return y


# speedup vs baseline: 1.2241x; 1.2241x over previous
import functools

import jax
import jax.numpy as jnp
from jax import lax
from jax.experimental import pallas as pl
from jax.experimental.pallas import tpu as pltpu
import jax.experimental.pallas.tpu_sc as plsc

HID = 70
H128 = 128
NC, NS = 2, 16
NW = NC * NS
CHUNK = 200
NCHUNKS = 256
NROUNDS = NCHUNKS // NW
B = 64


def _ceil_to(x, m):
    return (x + m - 1) // m * m


def _mm_body(x_ref, w_ref, b_ref, o_ref):
    o_ref[...] = (
        jnp.dot(x_ref[...], w_ref[...], preferred_element_type=jnp.float32)
        + b_ref[...]
    )


def _mm(x, w, b, bm=512):
    M, K = x.shape
    H = w.shape[1]
    Kp = _ceil_to(K, 128)
    Hp = _ceil_to(H, 128)
    Mp = _ceil_to(M, bm)
    xp = jnp.pad(x, ((0, Mp - M), (0, Kp - K)))
    wp = jnp.pad(w, ((0, Kp - K), (0, Hp - H)))
    bp = jnp.pad(b, (0, Hp - H)).reshape(1, Hp)
    out = pl.pallas_call(
        _mm_body,
        grid=(Mp // bm,),
        in_specs=[
            pl.BlockSpec((bm, Kp), lambda i: (i, 0)),
            pl.BlockSpec((Kp, Hp), lambda i: (0, 0)),
            pl.BlockSpec((1, Hp), lambda i: (0, 0)),
        ],
        out_specs=pl.BlockSpec((bm, Hp), lambda i: (i, 0)),
        out_shape=jax.ShapeDtypeStruct((Mp, Hp), jnp.float32),
    )(xp, wp, bp)
    return out[:M, :H]


def _proj_body(h_ref, w_ref, b_ref, a_ref, bb_ref, d_ref, e_ref):
    y = (
        jnp.dot(h_ref[...], w_ref[...], preferred_element_type=jnp.float32)
        + b_ref[...]
    )
    a_ref[...] = y[:, 0:128]
    bb_ref[...] = y[:, 128:256]
    d_ref[...] = y[:, 256:384]
    e_ref[...] = y[:, 384:512]


def _k_proj(h, wcat, bcat, bm=512):
    Np = h.shape[0]
    outs = pl.pallas_call(
        _proj_body,
        grid=(Np // bm,),
        in_specs=[
            pl.BlockSpec((bm, 128), lambda i: (i, 0)),
            pl.BlockSpec((128, 512), lambda i: (0, 0)),
            pl.BlockSpec((1, 512), lambda i: (0, 0)),
        ],
        out_specs=[pl.BlockSpec((bm, 128), lambda i: (i, 0))] * 4,
        out_shape=[jax.ShapeDtypeStruct((Np, 128), jnp.float32)] * 4,
    )(h, wcat, bcat)
    return outs


def _eupd_body(ets_ref, e_ref, gs_ref, gb_ref, cw_ref, cb_ref,
               enew_ref, ce_ref):
    x = jnp.maximum(ets_ref[...] * gs_ref[...] + gb_ref[...], 0.0)
    enew = e_ref[...] + x
    enew_ref[...] = enew
    ce_ref[...] = (
        jnp.dot(enew, cw_ref[...], preferred_element_type=jnp.float32)
        + cb_ref[...]
    )


def _k_edge_update(ets, e_prev, gs, gb, cw, cb, bm=1024):
    Ep = ets.shape[0]
    return pl.pallas_call(
        _eupd_body,
        grid=(Ep // bm,),
        in_specs=[
            pl.BlockSpec((bm, 128), lambda i: (i, 0)),
            pl.BlockSpec((bm, 128), lambda i: (i, 0)),
            pl.BlockSpec((1, 128), lambda i: (0, 0)),
            pl.BlockSpec((1, 128), lambda i: (0, 0)),
            pl.BlockSpec((128, 128), lambda i: (0, 0)),
            pl.BlockSpec((1, 128), lambda i: (0, 0)),
        ],
        out_specs=[pl.BlockSpec((bm, 128), lambda i: (i, 0))] * 2,
        out_shape=[jax.ShapeDtypeStruct((Ep, 128), jnp.float32)] * 2,
    )(ets, e_prev, gs, gb, cw, cb)


def _nstat_body(ah_ref, num_ref, den_ref, sn_ref, t_ref, s1_ref, s2_ref):
    t = (ah_ref[...] + num_ref[...] / (den_ref[...] + 1e-6)) * sn_ref[...]
    t_ref[...] = t

    @pl.when(pl.program_id(0) == 0)
    def _():
        s1_ref[...] = jnp.zeros_like(s1_ref)
        s2_ref[...] = jnp.zeros_like(s2_ref)

    s1_ref[...] += jnp.sum(t, axis=0, keepdims=True)
    s2_ref[...] += jnp.sum(t * t, axis=0, keepdims=True)


def _k_node_stats(ah, num, den, snn_b, bm=512):
    Np = ah.shape[0]
    return pl.pallas_call(
        _nstat_body,
        grid=(Np // bm,),
        in_specs=[pl.BlockSpec((bm, 128), lambda i: (i, 0))] * 4,
        out_specs=[
            pl.BlockSpec((bm, 128), lambda i: (i, 0)),
            pl.BlockSpec((1, 128), lambda i: (0, 0)),
            pl.BlockSpec((1, 128), lambda i: (0, 0)),
        ],
        out_shape=[
            jax.ShapeDtypeStruct((Np, 128), jnp.float32),
            jax.ShapeDtypeStruct((1, 128), jnp.float32),
            jax.ShapeDtypeStruct((1, 128), jnp.float32),
        ],
    )(ah, num, den, snn_b)


def _nupd_body(h_ref, t_ref, gs_ref, gb_ref, w_ref, b_ref,
               hn_ref, a_ref, bb_ref, d_ref, e_ref):
    hn = h_ref[...] + jnp.maximum(t_ref[...] * gs_ref[...] + gb_ref[...], 0.0)
    hn_ref[...] = hn
    y = (
        jnp.dot(hn, w_ref[...], preferred_element_type=jnp.float32)
        + b_ref[...]
    )
    a_ref[...] = y[:, 0:128]
    bb_ref[...] = y[:, 128:256]
    d_ref[...] = y[:, 256:384]
    e_ref[...] = y[:, 384:512]


def _k_node_update(h, t, gs, gb, wcat, bcat, bm=512):
    Np = h.shape[0]
    return pl.pallas_call(
        _nupd_body,
        grid=(Np // bm,),
        in_specs=[
            pl.BlockSpec((bm, 128), lambda i: (i, 0)),
            pl.BlockSpec((bm, 128), lambda i: (i, 0)),
            pl.BlockSpec((1, 128), lambda i: (0, 0)),
            pl.BlockSpec((1, 128), lambda i: (0, 0)),
            pl.BlockSpec((128, 512), lambda i: (0, 0)),
            pl.BlockSpec((1, 512), lambda i: (0, 0)),
        ],
        out_specs=[pl.BlockSpec((bm, 128), lambda i: (i, 0))] * 5,
        out_shape=[jax.ShapeDtypeStruct((Np, 128), jnp.float32)] * 5,
    )(h, t, gs, gb, wcat, bcat)


def _nfin_body(h_ref, t_ref, gs_ref, gb_ref, s_ref, *, bm, n_valid):
    hn = h_ref[...] + jnp.maximum(t_ref[...] * gs_ref[...] + gb_ref[...], 0.0)
    i = pl.program_id(0)
    rows = lax.broadcasted_iota(jnp.int32, (bm, 128), 0) + i * bm
    hn = jnp.where(rows < n_valid, hn, 0.0)

    @pl.when(i == 0)
    def _():
        s_ref[...] = jnp.zeros_like(s_ref)

    s_ref[...] += jnp.sum(hn, axis=0, keepdims=True)


def _k_node_final(h, t, gs, gb, n_valid, bm=512):
    Np = h.shape[0]
    return pl.pallas_call(
        functools.partial(_nfin_body, bm=bm, n_valid=n_valid),
        grid=(Np // bm,),
        in_specs=[
            pl.BlockSpec((bm, 128), lambda i: (i, 0)),
            pl.BlockSpec((bm, 128), lambda i: (i, 0)),
            pl.BlockSpec((1, 128), lambda i: (0, 0)),
            pl.BlockSpec((1, 128), lambda i: (0, 0)),
        ],
        out_specs=pl.BlockSpec((1, 128), lambda i: (0, 0)),
        out_shape=jax.ShapeDtypeStruct((1, 128), jnp.float32),
    )(h, t, gs, gb)


def _make_sc_edge(Np, Ep):
    mesh = plsc.VectorSubcoreMesh(core_axis_name="c", subcore_axis_name="s",
                                  num_cores=NC, num_subcores=NS)

    @functools.partial(
        pl.kernel,
        out_type=[
            jax.ShapeDtypeStruct((Ep, 128), jnp.float32),
            jax.ShapeDtypeStruct((Np, 128), jnp.float32),
            jax.ShapeDtypeStruct((Np, 128), jnp.float32),
            jax.ShapeDtypeStruct((NW, 256), jnp.float32),
        ],
        mesh=mesh,
        scratch_types=[
            pltpu.VMEM((CHUNK, 128), jnp.float32),
            pltpu.VMEM((CHUNK, 128), jnp.float32),
            pltpu.VMEM((B + 16,), jnp.int32),
            pltpu.VMEM((B + 16,), jnp.int32),
            pltpu.VMEM((B + 16,), jnp.float32),
            pltpu.VMEM((B, 128), jnp.float32),
            pltpu.VMEM((B + 16, 128), jnp.float32),
            pltpu.VMEM((B + 16, 128), jnp.float32),
            pltpu.VMEM((B + 16, 128), jnp.float32),
            pltpu.VMEM((B, 128), jnp.float32),
            pltpu.VMEM((B, 128), jnp.float32),
            pltpu.VMEM((256,), jnp.float32),
            pltpu.VMEM((NCHUNKS + 32,), jnp.int32),
            pltpu.SemaphoreType.DMA,
        ],
    )
    def sc_edge(dh_hbm, eh_hbm, bh_hbm, ce_hbm, src_hbm, dst_hbm, sn_hbm,
                offs_hbm, ets_hbm, num_hbm, den_hbm, stats_hbm,
                accn, accd, src_v, dst_v, sn_v, ce_v, dh_v, eh_v, bh_v,
                et_v, ets_v, stats_v, offs_v, sem):
        wid = lax.axis_index("s") * NC + lax.axis_index("c")
        pltpu.sync_copy(offs_hbm, offs_v)

        zero16 = jnp.zeros((16,), jnp.float32)
        stats0 = tuple(zero16 for _ in range(16))

        def do_chunk(c, stats):
            cbase = c * CHUNK
            start = offs_v[pl.ds(c, 16)][0]
            end = offs_v[pl.ds(c + 1, 16)][0]
            astart = (start // 8) * 8

            def zero_row(i, _):
                for j in range(8):
                    accn[i, pl.ds(j * 16, 16)] = zero16
                    accd[i, pl.ds(j * 16, 16)] = zero16
                return 0

            lax.fori_loop(0, CHUNK, zero_row, 0)

            nb = (end - astart + (B - 1)) // B

            def do_batch(i, stats):
                bbase = astart + i * B
                pltpu.sync_copy(src_hbm.at[pl.ds(bbase, B + 16)], src_v)
                pltpu.sync_copy(dst_hbm.at[pl.ds(bbase, B + 16)], dst_v)
                pltpu.sync_copy(sn_hbm.at[pl.ds(bbase, B + 16)], sn_v)
                pltpu.sync_copy(ce_hbm.at[pl.ds(bbase, B)], ce_v)
                pltpu.async_copy(dh_hbm.at[src_v], dh_v, sem).wait()
                pltpu.async_copy(eh_hbm.at[dst_v], eh_v, sem).wait()
                pltpu.async_copy(bh_hbm.at[src_v], bh_v, sem).wait()

                def phase_a(e, _):
                    snb = jnp.full((16,), sn_v[pl.ds(e, 16)][0], jnp.float32)
                    for j in range(8):
                        ds = pl.ds(j * 16, 16)
                        et = ce_v[e, ds] + dh_v[e, ds] + eh_v[e, ds]
                        et_v[e, ds] = et
                        ets_v[e, ds] = et * snb
                    return 0

                lax.fori_loop(0, B, phase_a, 0)
                pltpu.sync_copy(ets_v, ets_hbm.at[pl.ds(bbase, B)])

                lo = jnp.maximum(start - bbase, 0)
                hi = jnp.minimum(end - bbase, B)

                def phase_b(e, stats):
                    rel = dst_v[pl.ds(e, 16)][0] - cbase
                    new = []
                    for j in range(8):
                        ds = pl.ds(j * 16, 16)
                        et = et_v[e, ds]
                        sig = 1.0 / (1.0 + jnp.exp(-et))
                        plsc.addupdate(accn.at[rel, ds], sig * bh_v[e, ds])
                        plsc.addupdate(accd.at[rel, ds], sig)
                        x = ets_v[e, ds]
                        new.append(stats[j] + x)
                        new.append(stats[8 + j] + x * x)
                    s1 = new[0::2]
                    s2 = new[1::2]
                    return tuple(s1) + tuple(s2)

                return lax.fori_loop(lo, hi, phase_b, stats)

            stats = lax.fori_loop(0, nb, do_batch, stats)
            pltpu.sync_copy(accn, num_hbm.at[pl.ds(cbase, CHUNK)])
            pltpu.sync_copy(accd, den_hbm.at[pl.ds(cbase, CHUNK)])
            return stats

        stats = stats0
        for r in range(NROUNDS):
            stats = do_chunk(wid * NROUNDS + r, stats)

        for j in range(8):
            stats_v[pl.ds(j * 16, 16)] = stats[j]
            stats_v[pl.ds(128 + j * 16, 16)] = stats[8 + j]
        pltpu.sync_copy(stats_v, stats_hbm.at[wid])

    return sc_edge


def _pad_w(w):
    return jnp.pad(w, ((0, 128 - w.shape[0]), (0, 128 - w.shape[1])))


def _pad_b(b):
    return jnp.pad(b, (0, 128 - b.shape[0])).reshape(1, 128)


def _bn_coeffs(s1, s2, count, gamma, beta):
    m = s1 / count
    v = s2 / count - m * m
    inv = lax.rsqrt(v + 1e-5)
    gp = jnp.pad(gamma, (0, 128 - gamma.shape[0])).reshape(1, 128)
    bp = jnp.pad(beta, (0, 128 - beta.shape[0])).reshape(1, 128)
    gs = gp * inv
    gb = bp - m * gs
    return gs, gb


def kernel(nodes_feat, edges_feat, nodes_num_norm_sqrt, edges_num_norm_sqrt,
           edge_index, params):
    N = nodes_feat.shape[0]
    E = edge_index.shape[1]
    Np = NCHUNKS * CHUNK
    Ep = _ceil_to(E + 128, 1024)

    src = edge_index[0]
    dst = edge_index[1]

    perm = jnp.argsort(dst)
    dst_s = dst[perm]
    src_s = src[perm]
    sn_s = edges_num_norm_sqrt[:, 0][perm]
    ef_s = edges_feat[:, 0][perm]
    dst_sp = jnp.pad(dst_s, (0, Ep - E))
    src_sp = jnp.pad(src_s, (0, Ep - E))
    sn_sp = jnp.pad(sn_s, (0, Ep - E))
    offs = jnp.searchsorted(
        dst_s, jnp.arange(NCHUNKS + 1, dtype=jnp.int32) * CHUNK
    ).astype(jnp.int32)
    offs = jnp.pad(offs, (0, 31), constant_values=E)

    nf = jnp.pad(nodes_feat, ((0, Np - N), (0, 0)))
    h = _mm(nf, params['emb_h'][0], params['emb_h'][1])
    h = jnp.pad(h, ((0, 0), (0, 128 - h.shape[1])))
    we = jnp.pad(params['emb_e'][0][0], (0, 128 - HID))
    be = jnp.pad(params['emb_e'][1], (0, 128 - HID))
    e_cur = ef_s[:, None] * we[None, :] + be[None, :]
    e_cur = jnp.pad(e_cur, ((0, Ep - E), (0, 0)))

    snn_b = jnp.broadcast_to(
        jnp.pad(nodes_num_norm_sqrt, ((0, Np - N), (0, 0))), (Np, 128)
    )

    sc_edge = _make_sc_edge(Np, Ep)

    lps = params['layers']
    wcats = [
        jnp.concatenate(
            [_pad_w(lp[n][0]) for n in ['A', 'B', 'D', 'E']], axis=1)
        for lp in lps
    ]
    bcats = [
        jnp.concatenate(
            [_pad_b(lp[n][1]) for n in ['A', 'B', 'D', 'E']], axis=1)
        for lp in lps
    ]

    ah, bh, dh, eh = _k_proj(h, wcats[0], bcats[0])
    cw0 = _pad_w(lps[0]['C'][0])
    cb0 = _pad_b(lps[0]['C'][1])
    ce = _mm(e_cur, cw0, cb0, bm=1024)
    ce = jnp.pad(ce, ((0, 0), (0, 0)))

    nlayers = len(lps)
    hg = None
    for l in range(nlayers):
        lp = lps[l]
        ets, num, den, stats = sc_edge(
            dh, eh, bh, ce, src_sp, dst_sp, sn_sp, offs)
        t, s1, s2 = _k_node_stats(ah, num, den, snn_b)
        gs_h, gb_h = _bn_coeffs(s1, s2, float(N), lp['bn_h'][0],
                                lp['bn_h'][1])
        if l < nlayers - 1:
            h, ah, bh, dh, eh = _k_node_update(
                h, t, gs_h, gb_h, wcats[l + 1], bcats[l + 1])
            st = jnp.sum(stats, axis=0)
            gs_e, gb_e = _bn_coeffs(st[None, 0:128], st[None, 128:256],
                                    float(E), lp['bn_e'][0], lp['bn_e'][1])
            cw = _pad_w(lps[l + 1]['C'][0])
            cb = _pad_b(lps[l + 1]['C'][1])
            e_cur, ce = _k_edge_update(ets, e_cur, gs_e, gb_e, cw, cb)
        else:
            hsum = _k_node_final(h, t, gs_h, gb_h, N)
            hg = hsum / float(N)

    y = hg[:, :HID]
    nmlp = len(params['mlp'])
    for j, (w, b) in enumerate(params['mlp']):
        y = _mm(y, w, b, bm=8)
        if j < nmlp - 1:
            y = jax.nn.relu(y)
    return y


# speedup vs baseline: 1.9592x; 1.6005x over previous
import functools

import jax
import jax.numpy as jnp
from jax import lax
from jax.experimental import pallas as pl
from jax.experimental.pallas import tpu as pltpu
import jax.experimental.pallas.tpu_sc as plsc

HID = 70
H128 = 128
NC, NS = 2, 16
NW = NC * NS
CHUNK = 112
NCHUNKS = 448
NROUNDS = NCHUNKS // NW
B = 128


def _ceil_to(x, m):
    return (x + m - 1) // m * m


def _mm_body(x_ref, w_ref, b_ref, o_ref):
    o_ref[...] = (
        jnp.dot(x_ref[...], w_ref[...], preferred_element_type=jnp.float32)
        + b_ref[...]
    )


def _mm(x, w, b, bm=512):
    M, K = x.shape
    H = w.shape[1]
    Kp = _ceil_to(K, 128)
    Hp = _ceil_to(H, 128)
    Mp = _ceil_to(M, bm)
    xp = jnp.pad(x, ((0, Mp - M), (0, Kp - K)))
    wp = jnp.pad(w, ((0, Kp - K), (0, Hp - H)))
    bp = jnp.pad(b, (0, Hp - H)).reshape(1, Hp)
    out = pl.pallas_call(
        _mm_body,
        grid=(Mp // bm,),
        in_specs=[
            pl.BlockSpec((bm, Kp), lambda i: (i, 0)),
            pl.BlockSpec((Kp, Hp), lambda i: (0, 0)),
            pl.BlockSpec((1, Hp), lambda i: (0, 0)),
        ],
        out_specs=pl.BlockSpec((bm, Hp), lambda i: (i, 0)),
        out_shape=jax.ShapeDtypeStruct((Mp, Hp), jnp.float32),
    )(xp, wp, bp)
    return out[:M, :H]


def _proj_body(h_ref, w_ref, b_ref, a_ref, bb_ref, d_ref, e_ref):
    y = (
        jnp.dot(h_ref[...], w_ref[...], preferred_element_type=jnp.float32)
        + b_ref[...]
    )
    a_ref[...] = y[:, 0:128]
    bb_ref[...] = y[:, 128:256]
    d_ref[...] = y[:, 256:384]
    e_ref[...] = y[:, 384:512]


def _k_proj(h, wcat, bcat, bm=512):
    Np = h.shape[0]
    outs = pl.pallas_call(
        _proj_body,
        grid=(Np // bm,),
        in_specs=[
            pl.BlockSpec((bm, 128), lambda i: (i, 0)),
            pl.BlockSpec((128, 512), lambda i: (0, 0)),
            pl.BlockSpec((1, 512), lambda i: (0, 0)),
        ],
        out_specs=[pl.BlockSpec((bm, 128), lambda i: (i, 0))] * 4,
        out_shape=[jax.ShapeDtypeStruct((Np, 128), jnp.float32)] * 4,
    )(h, wcat, bcat)
    return outs


def _eupd_body(ets_ref, e_ref, gs_ref, gb_ref, cw_ref, cb_ref,
               enew_ref, ce_ref):
    x = jnp.maximum(ets_ref[...] * gs_ref[...] + gb_ref[...], 0.0)
    enew = e_ref[...] + x
    enew_ref[...] = enew
    ce_ref[...] = (
        jnp.dot(enew, cw_ref[...], preferred_element_type=jnp.float32)
        + cb_ref[...]
    )


def _k_edge_update(ets, e_prev, gs, gb, cw, cb, bm=1024):
    Ep = ets.shape[0]
    return pl.pallas_call(
        _eupd_body,
        grid=(Ep // bm,),
        in_specs=[
            pl.BlockSpec((bm, 128), lambda i: (i, 0)),
            pl.BlockSpec((bm, 128), lambda i: (i, 0)),
            pl.BlockSpec((1, 128), lambda i: (0, 0)),
            pl.BlockSpec((1, 128), lambda i: (0, 0)),
            pl.BlockSpec((128, 128), lambda i: (0, 0)),
            pl.BlockSpec((1, 128), lambda i: (0, 0)),
        ],
        out_specs=[pl.BlockSpec((bm, 128), lambda i: (i, 0))] * 2,
        out_shape=[jax.ShapeDtypeStruct((Ep, 128), jnp.float32)] * 2,
    )(ets, e_prev, gs, gb, cw, cb)


def _nstat_body(ah_ref, num_ref, den_ref, sn_ref, t_ref, s1_ref, s2_ref):
    t = (ah_ref[...] + num_ref[...] / (den_ref[...] + 1e-6)) * sn_ref[...]
    t_ref[...] = t

    @pl.when(pl.program_id(0) == 0)
    def _():
        s1_ref[...] = jnp.zeros_like(s1_ref)
        s2_ref[...] = jnp.zeros_like(s2_ref)

    s1_ref[...] += jnp.sum(t, axis=0, keepdims=True)
    s2_ref[...] += jnp.sum(t * t, axis=0, keepdims=True)


def _k_node_stats(ah, num, den, snn_b, bm=512):
    Np = ah.shape[0]
    return pl.pallas_call(
        _nstat_body,
        grid=(Np // bm,),
        in_specs=[pl.BlockSpec((bm, 128), lambda i: (i, 0))] * 4,
        out_specs=[
            pl.BlockSpec((bm, 128), lambda i: (i, 0)),
            pl.BlockSpec((1, 128), lambda i: (0, 0)),
            pl.BlockSpec((1, 128), lambda i: (0, 0)),
        ],
        out_shape=[
            jax.ShapeDtypeStruct((Np, 128), jnp.float32),
            jax.ShapeDtypeStruct((1, 128), jnp.float32),
            jax.ShapeDtypeStruct((1, 128), jnp.float32),
        ],
    )(ah, num, den, snn_b)


def _nupd_body(h_ref, t_ref, gs_ref, gb_ref, w_ref, b_ref,
               hn_ref, a_ref, bb_ref, d_ref, e_ref):
    hn = h_ref[...] + jnp.maximum(t_ref[...] * gs_ref[...] + gb_ref[...], 0.0)
    hn_ref[...] = hn
    y = (
        jnp.dot(hn, w_ref[...], preferred_element_type=jnp.float32)
        + b_ref[...]
    )
    a_ref[...] = y[:, 0:128]
    bb_ref[...] = y[:, 128:256]
    d_ref[...] = y[:, 256:384]
    e_ref[...] = y[:, 384:512]


def _k_node_update(h, t, gs, gb, wcat, bcat, bm=512):
    Np = h.shape[0]
    return pl.pallas_call(
        _nupd_body,
        grid=(Np // bm,),
        in_specs=[
            pl.BlockSpec((bm, 128), lambda i: (i, 0)),
            pl.BlockSpec((bm, 128), lambda i: (i, 0)),
            pl.BlockSpec((1, 128), lambda i: (0, 0)),
            pl.BlockSpec((1, 128), lambda i: (0, 0)),
            pl.BlockSpec((128, 512), lambda i: (0, 0)),
            pl.BlockSpec((1, 512), lambda i: (0, 0)),
        ],
        out_specs=[pl.BlockSpec((bm, 128), lambda i: (i, 0))] * 5,
        out_shape=[jax.ShapeDtypeStruct((Np, 128), jnp.float32)] * 5,
    )(h, t, gs, gb, wcat, bcat)


def _nfin_body(h_ref, t_ref, gs_ref, gb_ref, s_ref, *, bm, n_valid):
    hn = h_ref[...] + jnp.maximum(t_ref[...] * gs_ref[...] + gb_ref[...], 0.0)
    i = pl.program_id(0)
    rows = lax.broadcasted_iota(jnp.int32, (bm, 128), 0) + i * bm
    hn = jnp.where(rows < n_valid, hn, 0.0)

    @pl.when(i == 0)
    def _():
        s_ref[...] = jnp.zeros_like(s_ref)

    s_ref[...] += jnp.sum(hn, axis=0, keepdims=True)


def _k_node_final(h, t, gs, gb, n_valid, bm=512):
    Np = h.shape[0]
    return pl.pallas_call(
        functools.partial(_nfin_body, bm=bm, n_valid=n_valid),
        grid=(Np // bm,),
        in_specs=[
            pl.BlockSpec((bm, 128), lambda i: (i, 0)),
            pl.BlockSpec((bm, 128), lambda i: (i, 0)),
            pl.BlockSpec((1, 128), lambda i: (0, 0)),
            pl.BlockSpec((1, 128), lambda i: (0, 0)),
        ],
        out_specs=pl.BlockSpec((1, 128), lambda i: (0, 0)),
        out_shape=jax.ShapeDtypeStruct((1, 128), jnp.float32),
    )(h, t, gs, gb)


def _make_sc_edge(Np, Ep, store_ets):
    mesh = plsc.VectorSubcoreMesh(core_axis_name="c", subcore_axis_name="s",
                                  num_cores=NC, num_subcores=NS)
    NJ = 5

    if store_ets:
        outs = [
            jax.ShapeDtypeStruct((Ep, 128), jnp.float32),
            jax.ShapeDtypeStruct((Np, 128), jnp.float32),
            jax.ShapeDtypeStruct((Np, 128), jnp.float32),
            jax.ShapeDtypeStruct((NW, 256), jnp.float32),
        ]
    else:
        outs = [
            jax.ShapeDtypeStruct((Np, 128), jnp.float32),
            jax.ShapeDtypeStruct((Np, 128), jnp.float32),
        ]

    @functools.partial(
        pl.kernel,
        out_type=outs,
        mesh=mesh,
        scratch_types=[
            pltpu.VMEM((CHUNK, 128), jnp.float32),
            pltpu.VMEM((CHUNK, 128), jnp.float32),
            pltpu.VMEM((B,), jnp.int32),
            pltpu.VMEM((B,), jnp.int32),
            pltpu.VMEM((B + 16,), jnp.int32),
            pltpu.VMEM((B + 16,), jnp.float32),
            pltpu.VMEM((B, 128), jnp.float32),
            pltpu.VMEM((B, 128), jnp.float32),
            pltpu.VMEM((B, 128), jnp.float32),
            pltpu.VMEM((B, 128), jnp.float32),
            pltpu.VMEM((B, 128), jnp.float32),
            pltpu.VMEM((256,), jnp.float32),
            pltpu.VMEM((NCHUNKS + 32,), jnp.int32),
            pltpu.SemaphoreType.DMA,
            pltpu.SemaphoreType.DMA,
        ],
    )
    def sc_edge(*args):
        if store_ets:
            (dh_hbm, eh_hbm, bh_hbm, ce_hbm, src_hbm, dst_hbm, sn_hbm,
             offs_hbm, ets_hbm, num_hbm, den_hbm, stats_hbm,
             accn, accd, srcg, dstg, dsts, sns, ce_v, dh_v, eh_v, bh_v,
             ets_v, stats_v, offs_v, sem, isem) = args
        else:
            (dh_hbm, eh_hbm, bh_hbm, ce_hbm, src_hbm, dst_hbm, sn_hbm,
             offs_hbm, num_hbm, den_hbm,
             accn, accd, srcg, dstg, dsts, sns, ce_v, dh_v, eh_v, bh_v,
             ets_v, stats_v, offs_v, sem, isem) = args
        wid = lax.axis_index("s") * NC + lax.axis_index("c")
        pltpu.sync_copy(offs_hbm, offs_v)

        zero16 = jnp.zeros((16,), jnp.float32)
        stats0 = tuple(zero16 for _ in range(2 * NJ))

        def zero_pad_row(i, _):
            for j in range(NJ, 8):
                ds = pl.ds(j * 16, 16)
                accn[i, ds] = zero16
                accd[i, ds] = zero16
            return 0

        lax.fori_loop(0, CHUNK, zero_pad_row, 0)
        if store_ets:
            def zero_ets_row(i, _):
                for j in range(NJ, 8):
                    ets_v[i, pl.ds(j * 16, 16)] = zero16
                return 0

            lax.fori_loop(0, B, zero_ets_row, 0)

        def do_chunk(r, stats):
            c = wid * NROUNDS + r
            cbase = c * CHUNK
            start = offs_v[pl.ds(c, 16)][0]
            end = offs_v[pl.ds(c + 1, 16)][0]
            astart = (start // 8) * 8

            def zero_row(i, _):
                for j in range(NJ):
                    ds = pl.ds(j * 16, 16)
                    accn[i, ds] = zero16
                    accd[i, ds] = zero16
                return 0

            lax.fori_loop(0, CHUNK, zero_row, 0)

            nb = (end - astart + (B - 1)) // B

            def do_batch(i, stats):
                bbase = astart + i * B
                d_src = pltpu.async_copy(
                    src_hbm.at[pl.ds(bbase, B)], srcg, isem)
                d_dst = pltpu.async_copy(
                    dst_hbm.at[pl.ds(bbase, B)], dstg, isem)
                d_dsts = pltpu.async_copy(
                    dst_hbm.at[pl.ds(bbase, B + 16)], dsts, sem)
                d_sns = pltpu.async_copy(
                    sn_hbm.at[pl.ds(bbase, B + 16)], sns, sem)
                d_ce = pltpu.async_copy(
                    ce_hbm.at[pl.ds(bbase, B)], ce_v, sem)
                d_src.wait()
                d_dst.wait()
                g1 = pltpu.async_copy(dh_hbm.at[srcg], dh_v, sem)
                g2 = pltpu.async_copy(eh_hbm.at[dstg], eh_v, sem)
                g3 = pltpu.async_copy(bh_hbm.at[srcg], bh_v, sem)
                d_dsts.wait()
                d_sns.wait()
                d_ce.wait()
                g1.wait()
                g2.wait()
                g3.wait()

                lo = jnp.maximum(start - bbase, 0)
                hi = jnp.minimum(end - bbase, B)

                if store_ets:
                    def phase_a(e, _):
                        snb = jnp.full((16,), sns[pl.ds(e, 16)][0],
                                       jnp.float32)
                        for j in range(NJ):
                            ds = pl.ds(j * 16, 16)
                            et = ce_v[e, ds] + dh_v[e, ds] + eh_v[e, ds]
                            ets_v[e, ds] = et * snb
                        return 0

                    lax.fori_loop(0, B, phase_a, 0)
                    pltpu.sync_copy(ets_v, ets_hbm.at[pl.ds(bbase, B)])

                def phase_b(e, stats):
                    rel = dsts[pl.ds(e, 16)][0] - cbase
                    new = []
                    for j in range(NJ):
                        ds = pl.ds(j * 16, 16)
                        et = ce_v[e, ds] + dh_v[e, ds] + eh_v[e, ds]
                        sig = 1.0 / (1.0 + jnp.exp(-et))
                        plsc.addupdate(accn.at[rel, ds], sig * bh_v[e, ds])
                        plsc.addupdate(accd.at[rel, ds], sig)
                        if store_ets:
                            x = ets_v[e, ds]
                            new.append(stats[j] + x)
                            new.append(stats[NJ + j] + x * x)
                    if store_ets:
                        return tuple(new[0::2]) + tuple(new[1::2])
                    return stats

                return lax.fori_loop(lo, hi, phase_b, stats)

            stats = lax.fori_loop(0, nb, do_batch, stats)
            pltpu.sync_copy(accn, num_hbm.at[pl.ds(cbase, CHUNK)])
            pltpu.sync_copy(accd, den_hbm.at[pl.ds(cbase, CHUNK)])
            return stats

        stats = lax.fori_loop(0, NROUNDS, do_chunk, stats0)

        if store_ets:
            for j in range(NJ):
                stats_v[pl.ds(j * 16, 16)] = stats[j]
                stats_v[pl.ds(128 + j * 16, 16)] = stats[NJ + j]
            for j in range(NJ, 8):
                stats_v[pl.ds(j * 16, 16)] = zero16
                stats_v[pl.ds(128 + j * 16, 16)] = zero16
            pltpu.sync_copy(stats_v, stats_hbm.at[wid])

    return sc_edge


def _pad_w(w):
    return jnp.pad(w, ((0, 128 - w.shape[0]), (0, 128 - w.shape[1])))


def _pad_b(b):
    return jnp.pad(b, (0, 128 - b.shape[0])).reshape(1, 128)


def _bn_coeffs(s1, s2, count, gamma, beta):
    m = s1 / count
    v = s2 / count - m * m
    inv = lax.rsqrt(v + 1e-5)
    gp = jnp.pad(gamma, (0, 128 - gamma.shape[0])).reshape(1, 128)
    bp = jnp.pad(beta, (0, 128 - beta.shape[0])).reshape(1, 128)
    gs = gp * inv
    gb = bp - m * gs
    return gs, gb


def kernel(nodes_feat, edges_feat, nodes_num_norm_sqrt, edges_num_norm_sqrt,
           edge_index, params):
    N = nodes_feat.shape[0]
    E = edge_index.shape[1]
    Np = NCHUNKS * CHUNK
    Ep = _ceil_to(E + 128, 1024)

    src = edge_index[0]
    dst = edge_index[1]

    perm = jnp.argsort(dst)
    dst_s = dst[perm]
    src_s = src[perm]
    sn_s = edges_num_norm_sqrt[:, 0][perm]
    ef_s = edges_feat[:, 0][perm]
    dst_sp = jnp.pad(dst_s, (0, Ep - E))
    src_sp = jnp.pad(src_s, (0, Ep - E))
    sn_sp = jnp.pad(sn_s, (0, Ep - E))
    offs = jnp.searchsorted(
        dst_s, jnp.arange(NCHUNKS + 1, dtype=jnp.int32) * CHUNK
    ).astype(jnp.int32)
    offs = jnp.pad(offs, (0, 31), constant_values=E)

    nf = jnp.pad(nodes_feat, ((0, Np - N), (0, 0)))
    h = _mm(nf, params['emb_h'][0], params['emb_h'][1])
    h = jnp.pad(h, ((0, 0), (0, 128 - h.shape[1])))
    we = jnp.pad(params['emb_e'][0][0], (0, 128 - HID))
    be = jnp.pad(params['emb_e'][1], (0, 128 - HID))
    e_cur = ef_s[:, None] * we[None, :] + be[None, :]
    e_cur = jnp.pad(e_cur, ((0, Ep - E), (0, 0)))

    snn_b = jnp.broadcast_to(
        jnp.pad(nodes_num_norm_sqrt, ((0, Np - N), (0, 0))), (Np, 128)
    )

    sc_edge = _make_sc_edge(Np, Ep, True)
    sc_edge_last = _make_sc_edge(Np, Ep, False)

    lps = params['layers']
    wcats = [
        jnp.concatenate(
            [_pad_w(lp[n][0]) for n in ['A', 'B', 'D', 'E']], axis=1)
        for lp in lps
    ]
    bcats = [
        jnp.concatenate(
            [_pad_b(lp[n][1]) for n in ['A', 'B', 'D', 'E']], axis=1)
        for lp in lps
    ]

    ah, bh, dh, eh = _k_proj(h, wcats[0], bcats[0])
    cw0 = _pad_w(lps[0]['C'][0])
    cb0 = _pad_b(lps[0]['C'][1])
    ce = _mm(e_cur, cw0, cb0, bm=1024)
    ce = jnp.pad(ce, ((0, 0), (0, 0)))

    nlayers = len(lps)
    hg = None
    for l in range(nlayers):
        lp = lps[l]
        if l < nlayers - 1:
            ets, num, den, stats = sc_edge(
                dh, eh, bh, ce, src_sp, dst_sp, sn_sp, offs)
        else:
            num, den = sc_edge_last(
                dh, eh, bh, ce, src_sp, dst_sp, sn_sp, offs)
        t, s1, s2 = _k_node_stats(ah, num, den, snn_b)
        gs_h, gb_h = _bn_coeffs(s1, s2, float(N), lp['bn_h'][0],
                                lp['bn_h'][1])
        if l < nlayers - 1:
            h, ah, bh, dh, eh = _k_node_update(
                h, t, gs_h, gb_h, wcats[l + 1], bcats[l + 1])
            st = jnp.sum(stats, axis=0)
            gs_e, gb_e = _bn_coeffs(st[None, 0:128], st[None, 128:256],
                                    float(E), lp['bn_e'][0], lp['bn_e'][1])
            cw = _pad_w(lps[l + 1]['C'][0])
            cb = _pad_b(lps[l + 1]['C'][1])
            e_cur, ce = _k_edge_update(ets, e_cur, gs_e, gb_e, cw, cb)
        else:
            hsum = _k_node_final(h, t, gs_h, gb_h, N)
            hg = hsum / float(N)

    y = hg[:, :HID]
    nmlp = len(params['mlp'])
    for j, (w, b) in enumerate(params['mlp']):
        y = _mm(y, w, b, bm=8)
        if j < nmlp - 1:
            y = jax.nn.relu(y)
    return y
